# baseline (speedup 1.0000x reference)
import functools

import jax
import jax.numpy as jnp
from jax import lax
from jax.experimental import pallas as pl
from jax.experimental.pallas import tpu as pltpu
from jax.experimental.pallas import tpu_sc as plsc

NCROPS = 2
NCLS = 100000
FDIM = 128
HALF = 64
NB = 16384
MOM = 0.9

_NTILE = 16
_CHUNK = NB // _NTILE
_NJ = _CHUNK // 128


def _sc_mesh():
    return plsc.VectorSubcoreMesh(core_axis_name="c", subcore_axis_name="s")


@functools.partial(
    pl.kernel,
    out_type=(
        jax.ShapeDtypeStruct((NB,), jnp.int32),
        jax.ShapeDtypeStruct((NB,), jnp.float32),
    ),
    mesh=_sc_mesh(),
    scratch_types=[
        pltpu.VMEM_SHARED((NCLS,), jnp.int32),
        pltpu.VMEM_SHARED((NB,), jnp.float32),
        pltpu.VMEM((_NJ, 128), jnp.int32),
        pltpu.VMEM((_NJ, 128), jnp.int32),
        pltpu.VMEM((_NJ, 128), jnp.int32),
        pltpu.VMEM((128,), jnp.float32),
        pltpu.VMEM((128,), jnp.float32),
    ],
)
def _k1(l_ref, rep_out, cnt_out, rep_s, cnt_s, lbuf, idbuf, repbuf, ones, zeros):
    cid = lax.axis_index("c")
    sid = lax.axis_index("s")
    base = sid * _CHUNK
    for j in range(_NJ):
        pltpu.sync_copy(l_ref.at[pl.ds(base + j * 128, 128)], lbuf.at[j])
    for k in range(8):
        ones[pl.ds(k * 16, 16)] = jnp.full((16,), 1.0, jnp.float32)
        zeros[pl.ds(k * 16, 16)] = jnp.zeros((16,), jnp.float32)
    for j in range(_NJ):
        for k in range(8):
            idbuf[j, pl.ds(k * 16, 16)] = (
                lax.iota(jnp.int32, 16) + (base + j * 128 + k * 16)
            )
    for j in range(_NJ):
        pltpu.sync_copy(zeros, cnt_s.at[pl.ds(base + j * 128, 128)])
    for j in range(_NJ):
        pltpu.sync_copy(idbuf.at[j], rep_s.at[lbuf.at[j]])
    plsc.subcore_barrier()
    for j in range(_NJ):
        pltpu.sync_copy(rep_s.at[lbuf.at[j]], repbuf.at[j])
    for j in range(_NJ):
        pltpu.sync_copy(ones, cnt_s.at[repbuf.at[j]], add=True)
    plsc.subcore_barrier()

    @pl.when(cid == 0)
    def _():
        for j in range(_NJ):
            pltpu.sync_copy(repbuf.at[j], rep_out.at[pl.ds(base + j * 128, 128)])
        pltpu.sync_copy(
            cnt_s.at[pl.ds(base, _CHUNK)], cnt_out.at[pl.ds(base, _CHUNK)]
        )


@functools.partial(
    pl.kernel,
    out_type=(
        jax.ShapeDtypeStruct((2, NB), jnp.float32),
        jax.ShapeDtypeStruct((2, NB), jnp.float32),
        jax.ShapeDtypeStruct((2, NB), jnp.float32),
    ),
    mesh=_sc_mesh(),
    compiler_params=pltpu.CompilerParams(use_tc_tiling_on_sc=False),
    scratch_types=[
        pltpu.VMEM_SHARED((NB, HALF), jnp.float32),
        pltpu.VMEM((_NJ, 128), jnp.int32),
        pltpu.VMEM((_NJ, 128), jnp.int32),
        pltpu.VMEM((128, HALF), jnp.float32),
        pltpu.VMEM((128, HALF), jnp.float32),
        pltpu.VMEM((128, HALF), jnp.float32),
        pltpu.VMEM((128, FDIM), jnp.float32),
        pltpu.VMEM((128, FDIM), jnp.float32),
        pltpu.VMEM((_CHUNK,), jnp.float32),
        pltpu.VMEM((_CHUNK,), jnp.float32),
        pltpu.VMEM((_CHUNK,), jnp.float32),
        pltpu.VMEM((_CHUNK,), jnp.float32),
        pltpu.SemaphoreType.DMA,
        pltpu.SemaphoreType.DMA,
        pltpu.SemaphoreType.DMA,
        pltpu.SemaphoreType.DMA,
        pltpu.SemaphoreType.DMA,
    ],
)
def _k2(x_ref, rep_ref, l_ref, cnt_ref, img_ref, skt_ref, a_out, b_out, c_out,
        acc_s, idxbuf, lbuf, xb0, xb1, zb, ib, kb,
        cbuf, abuf, bbuf, c2buf, sem0, sem1, semz, semb, semg):
    cid = lax.axis_index("c")
    sid = lax.axis_index("s")
    row0 = sid * _CHUNK
    col0 = cid * HALF
    xbufs = (xb0, xb1)
    sems = (sem0, sem1)
    for j in range(_NJ):
        pltpu.async_copy(rep_ref.at[pl.ds(row0 + j * 128, 128)], idxbuf.at[j],
                         semz)
        pltpu.async_copy(l_ref.at[pl.ds(row0 + j * 128, 128)], lbuf.at[j],
                         semz)
    pltpu.async_copy(cnt_ref.at[pl.ds(row0, _CHUNK)], cbuf, semz)
    for r in range(128):
        for k in range(HALF // 16):
            zb[r, pl.ds(k * 16, 16)] = jnp.zeros((16,), jnp.float32)
    zcps = [pltpu.async_copy(zb, acc_s.at[pl.ds(row0 + j * 128, 128)], semz)
            for j in range(_NJ)]
    for j in range(_NJ):
        pltpu.make_async_copy(rep_ref.at[pl.ds(row0 + j * 128, 128)],
                              idxbuf.at[j], semz).wait()
        pltpu.make_async_copy(l_ref.at[pl.ds(row0 + j * 128, 128)],
                              lbuf.at[j], semz).wait()
    pltpu.make_async_copy(cnt_ref.at[pl.ds(row0, _CHUNK)], cbuf, semz).wait()
    for c in zcps:
        c.wait()
    plsc.subcore_barrier()
    nchunk = NCROPS * _NJ

    def _src(i):
        crop, j = divmod(i, _NJ)
        return x_ref.at[pl.ds(crop * NB + row0 + j * 128, 128),
                        pl.ds(col0, HALF)]

    pltpu.async_copy(_src(0), xbufs[0], sems[0])
    for i in range(nchunk):
        if i + 1 < nchunk:
            pltpu.async_copy(_src(i + 1), xbufs[(i + 1) % 2], sems[(i + 1) % 2])
        pltpu.make_async_copy(_src(i), xbufs[i % 2], sems[i % 2]).wait()
        pltpu.sync_copy(xbufs[i % 2], acc_s.at[idxbuf.at[i % _NJ]], add=True)
    plsc.subcore_barrier()

    sbufs = (xb0, xb1)

    def _start_sb(b):
        pltpu.async_copy(acc_s.at[pl.ds(row0 + b * 128, 128)], sbufs[b % 2],
                         semb)

    def _wait_sb(b):
        pltpu.make_async_copy(acc_s.at[pl.ds(row0 + b * 128, 128)],
                              sbufs[b % 2], semb).wait()

    _start_sb(0)
    zero16 = jnp.zeros((16,), jnp.float32)
    lane = lax.iota(jnp.int32, 16)
    for b in range(_NJ):
        pltpu.async_copy(img_ref.at[lbuf.at[b]], ib, semg)
        pltpu.async_copy(skt_ref.at[lbuf.at[b]], kb, semg)
        if b + 1 < _NJ:
            _start_sb(b + 1)
        _wait_sb(b)
        pltpu.make_async_copy(img_ref.at[lbuf.at[b]], ib, semg).wait()
        pltpu.make_async_copy(skt_ref.at[lbuf.at[b]], kb, semg).wait()
        sb = sbufs[b % 2]

        def grp_body(g, _):
            o = b * 128 + g * 16
            cnt16 = cbuf[pl.ds(o, 16)]
            mfv = 0.05 / jnp.maximum(cnt16, 1.0)

            def row_body(ln, carry):
                m_a, m_b, m_c = carry
                rl = g * 16 + ln
                mf = mfv[jnp.full((16,), ln, jnp.int32)]
                acc_a = zero16
                acc_b = zero16
                acc_c = zero16
                for c in range(HALF // 16):
                    sv = sb[rl, pl.ds(c * 16, 16)]
                    iv = ib[rl, pl.ds(col0 + c * 16, 16)]
                    kv = kb[rl, pl.ds(col0 + c * 16, 16)]
                    u = iv * MOM + sv * mf
                    acc_a = acc_a + u * u
                    acc_b = acc_b + u * kv
                    acc_c = acc_c + kv * kv
                for sh in (8, 4, 2, 1):
                    idx = (lane + sh) % 16
                    acc_a = acc_a + acc_a[idx]
                    acc_b = acc_b + acc_b[idx]
                    acc_c = acc_c + acc_c[idx]
                m = lane == ln
                return (
                    jnp.where(m, acc_a, m_a),
                    jnp.where(m, acc_b, m_b),
                    jnp.where(m, acc_c, m_c),
                )

            m_a, m_b, m_c = lax.fori_loop(
                0, 16, row_body, (zero16, zero16, zero16)
            )
            abuf[pl.ds(o, 16)] = m_a
            bbuf[pl.ds(o, 16)] = m_b
            c2buf[pl.ds(o, 16)] = m_c
            return 0

        lax.fori_loop(0, 8, grp_body, 0)
    pltpu.sync_copy(abuf, a_out.at[cid, pl.ds(row0, _CHUNK)])
    pltpu.sync_copy(bbuf, b_out.at[cid, pl.ds(row0, _CHUNK)])
    pltpu.sync_copy(c2buf, c_out.at[cid, pl.ds(row0, _CHUNK)])


def _k3_body(a_ref, b_ref, c_ref, cnt_ref, out_ref):
    av = a_ref[0:1, :] + a_ref[1:2, :]
    bv = b_ref[0:1, :] + b_ref[1:2, :]
    cv = c_ref[0:1, :] + c_ref[1:2, :]
    k = cnt_ref[...]
    valid = k > 0.0
    f = 1.0 - 2.0 * bv * lax.rsqrt(av) + cv
    fs = jnp.sum(jnp.where(valid, f, 0.0))
    ns = jnp.sum(jnp.where(valid, 1.0, 0.0))
    out_ref[0, 0] = fs / ns


_k3 = pl.pallas_call(
    _k3_body,
    out_specs=pl.BlockSpec(memory_space=pltpu.SMEM),
    out_shape=jax.ShapeDtypeStruct((1, 1), jnp.float32),
)


def kernel(x, l, center_img, center_skt):
    rep, cnt = _k1(l)
    pa, pb, pc = _k2(x, rep, l, cnt, center_img, center_skt)
    loss = _k3(pa, pb, pc, cnt.reshape(1, NB))
    return loss[0, 0]


# speedup vs baseline: 1.2373x; 1.2373x over previous
import functools

import jax
import jax.numpy as jnp
from jax import lax
from jax.experimental import pallas as pl
from jax.experimental.pallas import tpu as pltpu
from jax.experimental.pallas import tpu_sc as plsc

NCROPS = 2
NCLS = 100000
FDIM = 128
HALF = 64
NB = 16384
MOM = 0.9

_NTILE = 16
_CHUNK = NB // _NTILE
_NJ = _CHUNK // 128


def _sc_mesh():
    return plsc.VectorSubcoreMesh(core_axis_name="c", subcore_axis_name="s")


@functools.partial(
    pl.kernel,
    out_type=(
        jax.ShapeDtypeStruct((2, NB), jnp.float32),
        jax.ShapeDtypeStruct((2, NB), jnp.float32),
        jax.ShapeDtypeStruct((2, NB), jnp.float32),
        jax.ShapeDtypeStruct((NB,), jnp.float32),
    ),
    mesh=_sc_mesh(),
    compiler_params=pltpu.CompilerParams(use_tc_tiling_on_sc=False),
    scratch_types=[
        pltpu.VMEM_SHARED((NCLS,), jnp.int32),
        pltpu.VMEM_SHARED((NB,), jnp.float32),
        pltpu.VMEM_SHARED((NB, HALF), jnp.float32),
        pltpu.VMEM((_NJ, 128), jnp.int32),
        pltpu.VMEM((_NJ, 128), jnp.int32),
        pltpu.VMEM((128,), jnp.int32),
        pltpu.VMEM((128,), jnp.float32),
        pltpu.VMEM((128,), jnp.float32),
        pltpu.VMEM((128, HALF), jnp.float32),
        pltpu.VMEM((128, HALF), jnp.float32),
        pltpu.VMEM((128, FDIM), jnp.float32),
        pltpu.VMEM((128, FDIM), jnp.float32),
        pltpu.VMEM((_NJ, 128), jnp.float32),
        pltpu.VMEM((_CHUNK,), jnp.float32),
        pltpu.VMEM((_CHUNK,), jnp.float32),
        pltpu.VMEM((_CHUNK,), jnp.float32),
        pltpu.SemaphoreType.DMA,
        pltpu.SemaphoreType.DMA,
        pltpu.SemaphoreType.DMA,
        pltpu.SemaphoreType.DMA,
        pltpu.SemaphoreType.DMA,
    ],
)
def _k2(x_ref, l_ref, img_ref, skt_ref, a_out, b_out, c_out, cnt_out,
        rep_s, cnt_s, acc_s, lbuf, repbuf, idb, ones, zeros, xb0, xb1,
        ib, kb, cbuf, abuf, bbuf, c2buf, sem0, sem1, semz, semb, semg):
    cid = lax.axis_index("c")
    sid = lax.axis_index("s")
    row0 = sid * _CHUNK
    col0 = cid * HALF
    xbufs = (xb0, xb1)
    sems = (sem0, sem1)

    for j in range(_NJ):
        pltpu.async_copy(l_ref.at[pl.ds(row0 + j * 128, 128)], lbuf.at[j],
                         semz)
    for k in range(8):
        ones[pl.ds(k * 16, 16)] = jnp.full((16,), 1.0, jnp.float32)
        zeros[pl.ds(k * 16, 16)] = jnp.zeros((16,), jnp.float32)
    for r in range(128):
        for k in range(HALF // 16):
            xb0[r, pl.ds(k * 16, 16)] = jnp.zeros((16,), jnp.float32)
    zcps = [pltpu.async_copy(xb0, acc_s.at[pl.ds(row0 + j * 128, 128)], semz)
            for j in range(_NJ)]
    ccps = [pltpu.async_copy(zeros, cnt_s.at[pl.ds(row0 + j * 128, 128)], semz)
            for j in range(_NJ)]
    for j in range(_NJ):
        pltpu.make_async_copy(l_ref.at[pl.ds(row0 + j * 128, 128)],
                              lbuf.at[j], semz).wait()
    for c in zcps:
        c.wait()
    for c in ccps:
        c.wait()
    for j in range(_NJ):
        for k in range(8):
            idb[pl.ds(k * 16, 16)] = (
                lax.iota(jnp.int32, 16) + (row0 + j * 128 + k * 16)
            )
        pltpu.sync_copy(idb, rep_s.at[lbuf.at[j]])
    plsc.subcore_barrier()
    for j in range(_NJ):
        pltpu.sync_copy(rep_s.at[lbuf.at[j]], repbuf.at[j])
    for j in range(_NJ):
        pltpu.sync_copy(ones, cnt_s.at[repbuf.at[j]], add=True)
    nchunk = NCROPS * _NJ

    def _src(i):
        crop, j = divmod(i, _NJ)
        return x_ref.at[pl.ds(crop * NB + row0 + j * 128, 128),
                        pl.ds(col0, HALF)]

    pltpu.async_copy(_src(0), xbufs[0], sems[0])
    for i in range(nchunk):
        if i + 1 < nchunk:
            pltpu.async_copy(_src(i + 1), xbufs[(i + 1) % 2], sems[(i + 1) % 2])
        pltpu.make_async_copy(_src(i), xbufs[i % 2], sems[i % 2]).wait()
        pltpu.sync_copy(xbufs[i % 2], acc_s.at[repbuf.at[i % _NJ]], add=True)
    plsc.subcore_barrier()

    zero16 = jnp.zeros((16,), jnp.float32)
    lane = lax.iota(jnp.int32, 16)

    def _ridx(v):
        return repbuf.at[v // 2, pl.ds((v % 2) * 64, 64)]

    def _lidx(v):
        return lbuf.at[v // 2, pl.ds((v % 2) * 64, 64)]

    def _sb(v):
        return xbufs[v % 2].at[pl.ds(0, 64)]

    def _ib(v):
        return ib.at[pl.ds((v % 2) * 64, 64)]

    def _kb(v):
        return kb.at[pl.ds((v % 2) * 64, 64)]

    def _cb(v):
        return cbuf.at[v // 2, pl.ds((v % 2) * 64, 64)]

    def _start_blk(v):
        pltpu.async_copy(acc_s.at[_ridx(v)], _sb(v), semb)
        pltpu.async_copy(cnt_s.at[_ridx(v)], _cb(v), semb)
        pltpu.async_copy(img_ref.at[_lidx(v)], _ib(v), semg)
        pltpu.async_copy(skt_ref.at[_lidx(v)], _kb(v), semg)

    def _wait_blk(v):
        pltpu.make_async_copy(acc_s.at[_ridx(v)], _sb(v), semb).wait()
        pltpu.make_async_copy(cnt_s.at[_ridx(v)], _cb(v), semb).wait()
        pltpu.make_async_copy(img_ref.at[_lidx(v)], _ib(v), semg).wait()
        pltpu.make_async_copy(skt_ref.at[_lidx(v)], _kb(v), semg).wait()

    _start_blk(0)
    for v in range(2 * _NJ):
        if v + 1 < 2 * _NJ:
            _start_blk(v + 1)
        _wait_blk(v)
        h = (v % 2) * 64
        sb = xbufs[v % 2]

        def grp_body(g, _, v=v, h=h, sb=sb):
            o = v * 64 + g * 16
            cnt16 = cbuf[v // 2, pl.ds((v % 2) * 64 + g * 16, 16)]
            mfv = 0.05 / cnt16

            def row_body(ln, carry, g=g, h=h, sb=sb):
                m_a, m_b, m_c = carry
                rl = g * 16 + ln
                mf = mfv[jnp.full((16,), ln, jnp.int32)]
                acc_a = zero16
                acc_b = zero16
                acc_c = zero16
                for c in range(HALF // 16):
                    sv = sb[rl, pl.ds(c * 16, 16)]
                    iv = ib[h + rl, pl.ds(col0 + c * 16, 16)]
                    kv = kb[h + rl, pl.ds(col0 + c * 16, 16)]
                    u = iv * MOM + sv * mf
                    acc_a = acc_a + u * u
                    acc_b = acc_b + u * kv
                    acc_c = acc_c + kv * kv
                for sh in (8, 4, 2, 1):
                    idx = (lane + sh) % 16
                    acc_a = acc_a + acc_a[idx]
                    acc_b = acc_b + acc_b[idx]
                    acc_c = acc_c + acc_c[idx]
                m = lane == ln
                return (
                    jnp.where(m, acc_a, m_a),
                    jnp.where(m, acc_b, m_b),
                    jnp.where(m, acc_c, m_c),
                )

            m_a, m_b, m_c = lax.fori_loop(
                0, 16, row_body, (zero16, zero16, zero16)
            )
            abuf[pl.ds(o, 16)] = m_a
            bbuf[pl.ds(o, 16)] = m_b
            c2buf[pl.ds(o, 16)] = m_c
            return 0

        lax.fori_loop(0, 4, grp_body, 0)
    pltpu.sync_copy(abuf, a_out.at[cid, pl.ds(row0, _CHUNK)])
    pltpu.sync_copy(bbuf, b_out.at[cid, pl.ds(row0, _CHUNK)])
    pltpu.sync_copy(c2buf, c_out.at[cid, pl.ds(row0, _CHUNK)])

    @pl.when(cid == 0)
    def _():
        for j in range(_NJ):
            pltpu.sync_copy(cbuf.at[j], cnt_out.at[pl.ds(row0 + j * 128, 128)])


def _k3_body(a_ref, b_ref, c_ref, cnt_ref, out_ref):
    av = a_ref[0:1, :] + a_ref[1:2, :]
    bv = b_ref[0:1, :] + b_ref[1:2, :]
    cv = c_ref[0:1, :] + c_ref[1:2, :]
    w = 1.0 / cnt_ref[...]
    f = 1.0 - 2.0 * bv * lax.rsqrt(av) + cv
    out_ref[0, 0] = jnp.sum(f * w) / jnp.sum(w)


_k3 = pl.pallas_call(
    _k3_body,
    out_specs=pl.BlockSpec(memory_space=pltpu.SMEM),
    out_shape=jax.ShapeDtypeStruct((1, 1), jnp.float32),
)


def kernel(x, l, center_img, center_skt):
    pa, pb, pc, cnt = _k2(x, l, center_img, center_skt)
    loss = _k3(pa, pb, pc, cnt.reshape(1, NB))
    return loss[0, 0]


# speedup vs baseline: 1.3319x; 1.0764x over previous
import functools

import jax
import jax.numpy as jnp
from jax import lax
from jax.experimental import pallas as pl
from jax.experimental.pallas import tpu as pltpu
from jax.experimental.pallas import tpu_sc as plsc

NCROPS = 2
NCLS = 100000
FDIM = 128
HALF = 64
NB = 16384
MOM = 0.9

_NTILE = 16
_CHUNK = NB // _NTILE
_NJ = _CHUNK // 128


def _sc_mesh():
    return plsc.VectorSubcoreMesh(core_axis_name="c", subcore_axis_name="s")


@functools.partial(
    pl.kernel,
    out_type=(
        jax.ShapeDtypeStruct((2 * NB,), jnp.float32),
        jax.ShapeDtypeStruct((2 * NB,), jnp.float32),
        jax.ShapeDtypeStruct((2 * NB,), jnp.float32),
        jax.ShapeDtypeStruct((NB,), jnp.float32),
    ),
    mesh=_sc_mesh(),
    compiler_params=pltpu.CompilerParams(use_tc_tiling_on_sc=False),
    scratch_types=[
        pltpu.VMEM_SHARED((NCLS,), jnp.int32),
        pltpu.VMEM_SHARED((NB,), jnp.float32),
        pltpu.VMEM_SHARED((NB, HALF), jnp.float32),
        pltpu.VMEM((_NJ, 128), jnp.int32),
        pltpu.VMEM((_NJ, 128), jnp.int32),
        pltpu.VMEM((128,), jnp.int32),
        pltpu.VMEM((128,), jnp.float32),
        pltpu.VMEM((128,), jnp.float32),
        pltpu.VMEM((128, HALF), jnp.float32),
        pltpu.VMEM((128, HALF), jnp.float32),
        pltpu.VMEM((128, FDIM), jnp.float32),
        pltpu.VMEM((128, FDIM), jnp.float32),
        pltpu.VMEM((_NJ, 128), jnp.float32),
        pltpu.VMEM((_CHUNK,), jnp.float32),
        pltpu.VMEM((_CHUNK,), jnp.float32),
        pltpu.VMEM((_CHUNK,), jnp.float32),
        pltpu.SemaphoreType.DMA,
        pltpu.SemaphoreType.DMA,
        pltpu.SemaphoreType.DMA,
        pltpu.SemaphoreType.DMA,
        pltpu.SemaphoreType.DMA,
    ],
)
def _k2(x_ref, l_ref, img_ref, skt_ref, a_out, b_out, c_out, cnt_out,
        rep_s, cnt_s, acc_s, lbuf, repbuf, idb, ones, zeros, xb0, xb1,
        ib, kb, cbuf, abuf, bbuf, c2buf, sem0, sem1, semz, semb, semg):
    cid = lax.axis_index("c")
    sid = lax.axis_index("s")
    row0 = sid * _CHUNK
    col0 = cid * HALF
    xbufs = (xb0, xb1)
    sems = (sem0, sem1)

    for j in range(_NJ):
        pltpu.async_copy(l_ref.at[pl.ds(row0 + j * 128, 128)], lbuf.at[j],
                         semz)
    for k in range(8):
        ones[pl.ds(k * 16, 16)] = jnp.full((16,), 1.0, jnp.float32)
        zeros[pl.ds(k * 16, 16)] = jnp.zeros((16,), jnp.float32)
    for r in range(128):
        for k in range(HALF // 16):
            xb0[r, pl.ds(k * 16, 16)] = jnp.zeros((16,), jnp.float32)
    zcps = [pltpu.async_copy(xb0, acc_s.at[pl.ds(row0 + j * 128, 128)], semz)
            for j in range(_NJ)]
    ccps = [pltpu.async_copy(zeros, cnt_s.at[pl.ds(row0 + j * 128, 128)], semz)
            for j in range(_NJ)]
    for j in range(_NJ):
        pltpu.make_async_copy(l_ref.at[pl.ds(row0 + j * 128, 128)],
                              lbuf.at[j], semz).wait()
    for c in zcps:
        c.wait()
    for c in ccps:
        c.wait()
    for j in range(_NJ):
        for k in range(8):
            idb[pl.ds(k * 16, 16)] = (
                lax.iota(jnp.int32, 16) + (row0 + j * 128 + k * 16)
            )
        pltpu.sync_copy(idb, rep_s.at[lbuf.at[j]])
    plsc.subcore_barrier()
    for j in range(_NJ):
        pltpu.sync_copy(rep_s.at[lbuf.at[j]], repbuf.at[j])
    for j in range(_NJ):
        pltpu.sync_copy(ones, cnt_s.at[repbuf.at[j]], add=True)
    nchunk = NCROPS * _NJ

    def _src(i):
        crop, j = divmod(i, _NJ)
        return x_ref.at[pl.ds(crop * NB + row0 + j * 128, 128),
                        pl.ds(col0, HALF)]

    pltpu.async_copy(_src(0), xbufs[0], sems[0])
    for i in range(nchunk):
        if i + 1 < nchunk:
            pltpu.async_copy(_src(i + 1), xbufs[(i + 1) % 2], sems[(i + 1) % 2])
        pltpu.make_async_copy(_src(i), xbufs[i % 2], sems[i % 2]).wait()
        pltpu.sync_copy(xbufs[i % 2], acc_s.at[repbuf.at[i % _NJ]], add=True)
    plsc.subcore_barrier()

    zero16 = jnp.zeros((16,), jnp.float32)
    lane = lax.iota(jnp.int32, 16)

    def _ridx(v):
        return repbuf.at[v // 2, pl.ds((v % 2) * 64, 64)]

    def _lidx(v):
        return lbuf.at[v // 2, pl.ds((v % 2) * 64, 64)]

    def _sb(v):
        return xbufs[v % 2].at[pl.ds(0, 64)]

    def _ib(v):
        return ib.at[pl.ds((v % 2) * 64, 64)]

    def _kb(v):
        return kb.at[pl.ds((v % 2) * 64, 64)]

    def _cb(v):
        return cbuf.at[v // 2, pl.ds((v % 2) * 64, 64)]

    def _start_blk(v):
        pltpu.async_copy(acc_s.at[_ridx(v)], _sb(v), semb)
        pltpu.async_copy(cnt_s.at[_ridx(v)], _cb(v), semb)
        pltpu.async_copy(img_ref.at[_lidx(v)], _ib(v), semg)
        pltpu.async_copy(skt_ref.at[_lidx(v)], _kb(v), semg)

    def _wait_blk(v):
        pltpu.make_async_copy(acc_s.at[_ridx(v)], _sb(v), semb).wait()
        pltpu.make_async_copy(cnt_s.at[_ridx(v)], _cb(v), semb).wait()
        pltpu.make_async_copy(img_ref.at[_lidx(v)], _ib(v), semg).wait()
        pltpu.make_async_copy(skt_ref.at[_lidx(v)], _kb(v), semg).wait()

    _start_blk(0)
    for v in range(2 * _NJ):
        if v + 1 < 2 * _NJ:
            _start_blk(v + 1)
        _wait_blk(v)
        h = (v % 2) * 64
        sb = xbufs[v % 2]

        def grp_body(g, _, v=v, h=h, sb=sb):
            o = v * 64 + g * 16
            cnt16 = cbuf[v // 2, pl.ds((v % 2) * 64 + g * 16, 16)]
            mfv = 0.05 / cnt16

            def row_body(ln, carry, g=g, h=h, sb=sb):
                m_a, m_b, m_c = carry
                rl = g * 16 + ln
                mf = mfv[jnp.full((16,), ln, jnp.int32)]
                acc_a = zero16
                acc_b = zero16
                acc_c = zero16
                for c in range(HALF // 16):
                    sv = sb[rl, pl.ds(c * 16, 16)]
                    iv = ib[h + rl, pl.ds(col0 + c * 16, 16)]
                    kv = kb[h + rl, pl.ds(col0 + c * 16, 16)]
                    u = iv * MOM + sv * mf
                    acc_a = acc_a + u * u
                    acc_b = acc_b + u * kv
                    acc_c = acc_c + kv * kv
                for sh in (8, 4, 2, 1):
                    idx = (lane + sh) % 16
                    acc_a = acc_a + acc_a[idx]
                    acc_b = acc_b + acc_b[idx]
                    acc_c = acc_c + acc_c[idx]
                m = lane == ln
                return (
                    jnp.where(m, acc_a, m_a),
                    jnp.where(m, acc_b, m_b),
                    jnp.where(m, acc_c, m_c),
                )

            m_a, m_b, m_c = lax.fori_loop(
                0, 16, row_body, (zero16, zero16, zero16)
            )
            abuf[pl.ds(o, 16)] = m_a
            bbuf[pl.ds(o, 16)] = m_b
            c2buf[pl.ds(o, 16)] = m_c
            return 0

        lax.fori_loop(0, 4, grp_body, 0)
    obase = cid * NB + row0
    pltpu.sync_copy(abuf, a_out.at[pl.ds(obase, _CHUNK)])
    pltpu.sync_copy(bbuf, b_out.at[pl.ds(obase, _CHUNK)])
    pltpu.sync_copy(c2buf, c_out.at[pl.ds(obase, _CHUNK)])

    @pl.when(cid == 0)
    def _():
        for j in range(_NJ):
            pltpu.sync_copy(cbuf.at[j], cnt_out.at[pl.ds(row0 + j * 128, 128)])


def _k3_body(a_ref, b_ref, c_ref, cnt_ref, out_ref):
    av = a_ref[0:NB] + a_ref[NB:]
    bv = b_ref[0:NB] + b_ref[NB:]
    cv = c_ref[0:NB] + c_ref[NB:]
    w = 1.0 / cnt_ref[...]
    f = 1.0 - 2.0 * bv * lax.rsqrt(av) + cv
    out_ref[0, 0] = jnp.sum(f * w) / jnp.sum(w)


_k3 = pl.pallas_call(
    _k3_body,
    out_specs=pl.BlockSpec(memory_space=pltpu.SMEM),
    out_shape=jax.ShapeDtypeStruct((1, 1), jnp.float32),
)


def kernel(x, l, center_img, center_skt):
    pa, pb, pc, cnt = _k2(x, l, center_img, center_skt)
    loss = _k3(pa, pb, pc, cnt)
    return loss[0, 0]


# speedup vs baseline: 1.3384x; 1.0049x over previous
import functools

import jax
import jax.numpy as jnp
from jax import lax
from jax.experimental import pallas as pl
from jax.experimental.pallas import tpu as pltpu
from jax.experimental.pallas import tpu_sc as plsc

NCROPS = 2
NCLS = 100000
FDIM = 128
HALF = 64
NB = 16384
MOM = 0.9

_NTILE = 16
_CHUNK = NB // _NTILE
_NJ = _CHUNK // 128


def _sc_mesh():
    return plsc.VectorSubcoreMesh(core_axis_name="c", subcore_axis_name="s")


@functools.partial(
    pl.kernel,
    out_type=(
        jax.ShapeDtypeStruct((2 * NB,), jnp.float32),
        jax.ShapeDtypeStruct((2 * NB,), jnp.float32),
        jax.ShapeDtypeStruct((2 * NB,), jnp.float32),
        jax.ShapeDtypeStruct((NB,), jnp.float32),
    ),
    mesh=_sc_mesh(),
    compiler_params=pltpu.CompilerParams(use_tc_tiling_on_sc=False),
    scratch_types=[
        pltpu.VMEM_SHARED((NCLS,), jnp.int32),
        pltpu.VMEM_SHARED((NB,), jnp.float32),
        pltpu.VMEM_SHARED((NB, HALF), jnp.float32),
        pltpu.VMEM((_NJ, 128), jnp.int32),
        pltpu.VMEM((_NJ, 128), jnp.int32),
        pltpu.VMEM((128,), jnp.int32),
        pltpu.VMEM((128,), jnp.float32),
        pltpu.VMEM((128,), jnp.float32),
        pltpu.VMEM((128, HALF), jnp.float32),
        pltpu.VMEM((128, HALF), jnp.float32),
        pltpu.VMEM((128, FDIM), jnp.float32),
        pltpu.VMEM((128, FDIM), jnp.float32),
        pltpu.VMEM((_NJ, 128), jnp.float32),
        pltpu.VMEM((_CHUNK,), jnp.float32),
        pltpu.VMEM((_CHUNK,), jnp.float32),
        pltpu.VMEM((_CHUNK,), jnp.float32),
        pltpu.SemaphoreType.DMA,
        pltpu.SemaphoreType.DMA,
        pltpu.SemaphoreType.DMA,
        pltpu.SemaphoreType.DMA,
        pltpu.SemaphoreType.DMA,
        pltpu.SemaphoreType.DMA,
    ],
)
def _k2(x_ref, l_ref, img_ref, skt_ref, a_out, b_out, c_out, cnt_out,
        rep_s, cnt_s, acc_s, lbuf, repbuf, idb, ones, zeros, xb0, xb1,
        ib, kb, cbuf, abuf, bbuf, c2buf, sem0, sem1, semz, semg0, semg1,
        semc):
    cid = lax.axis_index("c")
    sid = lax.axis_index("s")
    row0 = sid * _CHUNK
    col0 = cid * HALF
    xbufs = (xb0, xb1)
    sems = (sem0, sem1)

    for j in range(_NJ):
        pltpu.async_copy(l_ref.at[pl.ds(row0 + j * 128, 128)], lbuf.at[j],
                         semz)
    for k in range(8):
        ones[pl.ds(k * 16, 16)] = jnp.full((16,), 1.0, jnp.float32)
        zeros[pl.ds(k * 16, 16)] = jnp.zeros((16,), jnp.float32)
    for r in range(128):
        for k in range(HALF // 16):
            xb0[r, pl.ds(k * 16, 16)] = jnp.zeros((16,), jnp.float32)
    zcps = [pltpu.async_copy(xb0, acc_s.at[pl.ds(row0 + j * 128, 128)], semz)
            for j in range(_NJ)]
    ccps = [pltpu.async_copy(zeros, cnt_s.at[pl.ds(row0 + j * 128, 128)], semz)
            for j in range(_NJ)]
    for j in range(_NJ):
        pltpu.make_async_copy(l_ref.at[pl.ds(row0 + j * 128, 128)],
                              lbuf.at[j], semz).wait()
    for c in zcps:
        c.wait()
    for c in ccps:
        c.wait()
    for j in range(_NJ):
        for k in range(8):
            idb[pl.ds(k * 16, 16)] = (
                lax.iota(jnp.int32, 16) + (row0 + j * 128 + k * 16)
            )
        pltpu.sync_copy(idb, rep_s.at[lbuf.at[j]])
    plsc.subcore_barrier()
    for j in range(_NJ):
        pltpu.sync_copy(rep_s.at[lbuf.at[j]], repbuf.at[j])
    for j in range(_NJ):
        pltpu.sync_copy(ones, cnt_s.at[repbuf.at[j]], add=True)
    nchunk = NCROPS * _NJ

    def _src(i):
        crop, j = divmod(i, _NJ)
        return x_ref.at[pl.ds(crop * NB + row0 + j * 128, 128),
                        pl.ds(col0, HALF)]

    gsems = (semg0, semg1)

    def _ridx(v):
        return repbuf.at[v // 2, pl.ds((v % 2) * 64, 64)]

    def _lidx(v):
        return lbuf.at[v // 2, pl.ds((v % 2) * 64, 64)]

    def _sb(v):
        return xbufs[v % 2].at[pl.ds(0, 64)]

    def _ib(v):
        return ib.at[pl.ds((v % 2) * 64, 64)]

    def _kb(v):
        return kb.at[pl.ds((v % 2) * 64, 64)]

    def _cb(v):
        return cbuf.at[v // 2, pl.ds((v % 2) * 64, 64)]

    def _start_g(v):
        pltpu.async_copy(img_ref.at[_lidx(v)], _ib(v), gsems[v % 2])
        pltpu.async_copy(skt_ref.at[_lidx(v)], _kb(v), gsems[v % 2])

    _start_g(0)
    _start_g(1)
    pltpu.async_copy(_src(0), xbufs[0], sems[0])
    for i in range(nchunk):
        if i + 1 < nchunk:
            pltpu.async_copy(_src(i + 1), xbufs[(i + 1) % 2], sems[(i + 1) % 2])
        pltpu.make_async_copy(_src(i), xbufs[i % 2], sems[i % 2]).wait()
        pltpu.sync_copy(xbufs[i % 2], acc_s.at[repbuf.at[i % _NJ]], add=True)
    plsc.subcore_barrier()

    zero16 = jnp.zeros((16,), jnp.float32)
    lane = lax.iota(jnp.int32, 16)

    def _start_sc(v):
        pltpu.async_copy(acc_s.at[_ridx(v)], _sb(v), sems[v % 2])
        pltpu.async_copy(cnt_s.at[_ridx(v)], _cb(v), sems[v % 2])

    def _wait_blk(v):
        pltpu.make_async_copy(acc_s.at[_ridx(v)], _sb(v), sems[v % 2]).wait()
        pltpu.make_async_copy(cnt_s.at[_ridx(v)], _cb(v), sems[v % 2]).wait()
        pltpu.make_async_copy(img_ref.at[_lidx(v)], _ib(v),
                              gsems[v % 2]).wait()
        pltpu.make_async_copy(skt_ref.at[_lidx(v)], _kb(v),
                              gsems[v % 2]).wait()

    _start_sc(0)
    for v in range(2 * _NJ):
        if v + 1 < 2 * _NJ:
            _start_sc(v + 1)
        _wait_blk(v)
        h = (v % 2) * 64
        sb = xbufs[v % 2]

        def grp_body(g, _, v=v, h=h, sb=sb):
            o = v * 64 + g * 16
            cnt16 = cbuf[v // 2, pl.ds((v % 2) * 64 + g * 16, 16)]
            mfv = 0.05 / cnt16

            def row_body(ln, carry, g=g, h=h, sb=sb):
                m_a, m_b, m_c = carry
                rl = g * 16 + ln
                mf = mfv[jnp.full((16,), ln, jnp.int32)]
                acc_a = zero16
                acc_b = zero16
                acc_c = zero16
                for c in range(HALF // 16):
                    sv = sb[rl, pl.ds(c * 16, 16)]
                    iv = ib[h + rl, pl.ds(col0 + c * 16, 16)]
                    kv = kb[h + rl, pl.ds(col0 + c * 16, 16)]
                    u = iv * MOM + sv * mf
                    acc_a = acc_a + u * u
                    acc_b = acc_b + u * kv
                    acc_c = acc_c + kv * kv
                for sh in (8, 4, 2, 1):
                    idx = (lane + sh) % 16
                    acc_a = acc_a + acc_a[idx]
                    acc_b = acc_b + acc_b[idx]
                    acc_c = acc_c + acc_c[idx]
                m = lane == ln
                return (
                    jnp.where(m, acc_a, m_a),
                    jnp.where(m, acc_b, m_b),
                    jnp.where(m, acc_c, m_c),
                )

            m_a, m_b, m_c = lax.fori_loop(
                0, 16, row_body, (zero16, zero16, zero16)
            )
            abuf[pl.ds(o, 16)] = m_a
            bbuf[pl.ds(o, 16)] = m_b
            c2buf[pl.ds(o, 16)] = m_c
            return 0

        lax.fori_loop(0, 4, grp_body, 0)
        if v + 2 < 2 * _NJ:
            _start_g(v + 2)
    obase = cid * NB + row0
    pltpu.sync_copy(abuf, a_out.at[pl.ds(obase, _CHUNK)])
    pltpu.sync_copy(bbuf, b_out.at[pl.ds(obase, _CHUNK)])
    pltpu.sync_copy(c2buf, c_out.at[pl.ds(obase, _CHUNK)])

    @pl.when(cid == 0)
    def _():
        for j in range(_NJ):
            pltpu.sync_copy(cbuf.at[j], cnt_out.at[pl.ds(row0 + j * 128, 128)])


def _k3_body(a_ref, b_ref, c_ref, cnt_ref, out_ref):
    av = a_ref[0:NB] + a_ref[NB:]
    bv = b_ref[0:NB] + b_ref[NB:]
    cv = c_ref[0:NB] + c_ref[NB:]
    w = 1.0 / cnt_ref[...]
    f = 1.0 - 2.0 * bv * lax.rsqrt(av) + cv
    out_ref[0, 0] = jnp.sum(f * w) / jnp.sum(w)


_k3 = pl.pallas_call(
    _k3_body,
    out_specs=pl.BlockSpec(memory_space=pltpu.SMEM),
    out_shape=jax.ShapeDtypeStruct((1, 1), jnp.float32),
)


def kernel(x, l, center_img, center_skt):
    pa, pb, pc, cnt = _k2(x, l, center_img, center_skt)
    loss = _k3(pa, pb, pc, cnt)
    return loss[0, 0]
